# Initial kernel scaffold; baseline (speedup 1.0000x reference)
import functools

import jax
import jax.numpy as jnp
from jax import lax
from jax.experimental import pallas as pl
from jax.experimental.pallas import tpu as pltpu
from jax.experimental.pallas import tpu_sc as plsc

N = 320000
D = 128
G = 2048
NC = 2
NS = 16
NW = NC * NS
SEG_PER_W = G // NW
T = 512
LANES = 16


def _wmr_body(h_hbm, pos_hbm, seg_hbm, table_hbm, offs_hbm, out_hbm,
              ht, segt, post, at, acc, den, table_v, offs_v):
    wid = lax.axis_index("s") * NC + lax.axis_index("c")
    g0 = wid * SEG_PER_W

    pltpu.sync_copy(table_hbm, table_v)
    pltpu.sync_copy(offs_hbm, offs_v)
    r0 = offs_v[wid]
    r1 = offs_v[wid + 1]

    zeros = jnp.zeros((LANES,), jnp.float32)

    def zrow(l, carry):
        for j in range(D // LANES):
            acc[l, pl.ds(j * LANES, LANES)] = zeros
        return carry

    lax.fori_loop(0, SEG_PER_W, zrow, 0)
    for j in range(SEG_PER_W // LANES):
        den[pl.ds(j * LANES, LANES)] = zeros

    t0 = r0 // T
    t1 = (r1 + T - 1) // T

    def tile_body(t, carry):
        base = t * T
        pltpu.sync_copy(h_hbm.at[pl.ds(base, T)], ht)
        pltpu.sync_copy(seg_hbm.at[pl.ds(base, T)], segt)
        pltpu.sync_copy(pos_hbm.at[pl.ds(base, T)], post)
        for j in range(T // LANES):
            pv = post[pl.ds(j * LANES, LANES)]
            at[pl.ds(j * LANES, LANES)] = plsc.load_gather(table_v, [pv])

        i_lo = jnp.maximum(r0 - base, 0)
        i_hi = jnp.minimum(r1 - base, T)

        def row_body(i, c):
            l = segt[i] - g0
            a = at[i]
            for j in range(D // LANES):
                plsc.addupdate(acc.at[l, pl.ds(j * LANES, LANES)],
                               a * ht[i, pl.ds(j * LANES, LANES)])
            den[l] = den[l] + a
            return c

        lax.fori_loop(i_lo, i_hi, row_body, 0)
        return carry

    lax.fori_loop(t0, t1, tile_body, 0)

    for j in range(SEG_PER_W // LANES):
        dv = den[pl.ds(j * LANES, LANES)]
        den[pl.ds(j * LANES, LANES)] = 1.0 / jnp.maximum(dv, 1e-12)

    def fin_row(l, carry):
        r = den[l]
        for j in range(D // LANES):
            acc[l, pl.ds(j * LANES, LANES)] = acc[l, pl.ds(j * LANES, LANES)] * r
        return carry

    lax.fori_loop(0, SEG_PER_W, fin_row, 0)
    pltpu.sync_copy(acc, out_hbm.at[pl.ds(g0, SEG_PER_W)])


_wmr = pl.kernel(
    _wmr_body,
    mesh=plsc.VectorSubcoreMesh(core_axis_name="c", subcore_axis_name="s"),
    out_type=jax.ShapeDtypeStruct((G, D), jnp.float32),
    scratch_types=[
        pltpu.VMEM((T, D), jnp.float32),
        pltpu.VMEM((T,), jnp.int32),
        pltpu.VMEM((T,), jnp.int32),
        pltpu.VMEM((T,), jnp.float32),
        pltpu.VMEM((SEG_PER_W, D), jnp.float32),
        pltpu.VMEM((SEG_PER_W,), jnp.float32),
        pltpu.VMEM((LANES,), jnp.float32),
        pltpu.VMEM((NW + 8,), jnp.int32),
    ],
)


def kernel(h, pos, segment_ids, pos_weight):
    table = jax.nn.softplus(pos_weight[:, 0].astype(jnp.float32))
    table = jnp.pad(table, (0, LANES - table.shape[0]))
    bounds = jnp.arange(NW + 1, dtype=jnp.int32) * SEG_PER_W
    offs = jnp.searchsorted(segment_ids, bounds, side="left").astype(jnp.int32)
    offs = jnp.pad(offs, (0, 7))
    return _wmr(h, pos, segment_ids, table, offs)


# speedup vs baseline: 2.2286x; 2.2286x over previous
import jax
import jax.numpy as jnp
from jax import lax
from jax.experimental import pallas as pl
from jax.experimental.pallas import tpu as pltpu
from jax.experimental.pallas import tpu_sc as plsc

N = 320000
D = 128
G = 2048
NC = 2
NS = 16
NW = NC * NS
SEG_PER_W = G // NW
T = 512
LANES = 16
NVR = D // LANES


def _wmr_body(h_hbm, pos_hbm, seg_hbm, table_hbm, offs_hbm, out_hbm,
              ht, segt, post, at, acc, den, table_v, offs_v):
    wid = lax.axis_index("s") * NC + lax.axis_index("c")
    g0 = wid * SEG_PER_W

    pltpu.sync_copy(table_hbm, table_v)
    pltpu.sync_copy(offs_hbm, offs_v)
    off_pair = offs_v[pl.ds(wid, LANES)]
    r0 = off_pair[0]
    r1 = off_pair[1]

    zeros = jnp.zeros((LANES,), jnp.float32)
    lane0 = jnp.where(lax.iota(jnp.int32, LANES) == 0, 1.0, 0.0)

    def zchunk(k, carry):
        acc[pl.ds(k * LANES, LANES)] = zeros
        return carry

    lax.fori_loop(0, SEG_PER_W * D // LANES, zchunk, 0)
    for j in range(SEG_PER_W // LANES):
        den[pl.ds(j * LANES, LANES)] = zeros

    t0 = r0 // T
    t1 = (r1 + T - 1) // T

    def tile_body(t, carry):
        base = t * T
        pltpu.sync_copy(h_hbm.at[pl.ds(base * D, T * D)], ht)
        pltpu.sync_copy(seg_hbm.at[pl.ds(base, T)], segt.at[pl.ds(0, T)])
        pltpu.sync_copy(pos_hbm.at[pl.ds(base, T)], post)
        for j in range(T // LANES):
            pv = post[pl.ds(j * LANES, LANES)]
            at[pl.ds(j * LANES, LANES)] = plsc.load_gather(table_v, [pv])

        i_lo = jnp.maximum(r0 - base, 0)
        i_hi = jnp.minimum(r1 - base, T)

        def row_body(i, c):
            l = segt[pl.ds(i, LANES)][0] - g0
            a = at[pl.ds(i, LANES)][0]
            for j in range(NVR):
                plsc.addupdate(acc.at[pl.ds(l * D + j * LANES, LANES)],
                               a * ht[pl.ds(i * D + j * LANES, LANES)])
            dv = den[pl.ds(l, LANES)]
            den[pl.ds(l, LANES)] = dv + a * lane0
            return c

        lax.fori_loop(i_lo, i_hi, row_body, 0)
        return carry

    lax.fori_loop(t0, t1, tile_body, 0)

    for j in range(SEG_PER_W // LANES):
        dv = den[pl.ds(j * LANES, LANES)]
        den[pl.ds(j * LANES, LANES)] = 1.0 / jnp.maximum(dv, 1e-12)

    def fin_row(l, carry):
        r = den[pl.ds(l, LANES)][0]
        for j in range(NVR):
            o = l * D + j * LANES
            acc[pl.ds(o, LANES)] = acc[pl.ds(o, LANES)] * r
        return carry

    lax.fori_loop(0, SEG_PER_W, fin_row, 0)
    pltpu.sync_copy(acc, out_hbm.at[pl.ds(g0 * D, SEG_PER_W * D)])


_wmr = pl.kernel(
    _wmr_body,
    mesh=plsc.VectorSubcoreMesh(core_axis_name="c", subcore_axis_name="s"),
    out_type=jax.ShapeDtypeStruct((G * D,), jnp.float32),
    compiler_params=pltpu.CompilerParams(needs_layout_passes=False),
    scratch_types=[
        pltpu.VMEM((T * D,), jnp.float32),
        pltpu.VMEM((T + LANES,), jnp.int32),
        pltpu.VMEM((T,), jnp.int32),
        pltpu.VMEM((T + LANES,), jnp.float32),
        pltpu.VMEM((SEG_PER_W * D,), jnp.float32),
        pltpu.VMEM((SEG_PER_W + LANES,), jnp.float32),
        pltpu.VMEM((LANES,), jnp.float32),
        pltpu.VMEM((NW + 1 + LANES - 1,), jnp.int32),
    ],
)


def kernel(h, pos, segment_ids, pos_weight):
    table = jax.nn.softplus(pos_weight[:, 0].astype(jnp.float32))
    table = jnp.pad(table, (0, LANES - table.shape[0]))
    bounds = jnp.arange(NW + 1, dtype=jnp.int32) * SEG_PER_W
    offs = jnp.searchsorted(segment_ids, bounds, side="left").astype(jnp.int32)
    offs = jnp.pad(offs, (0, LANES - 1))
    out = _wmr(h.reshape(-1), pos, segment_ids, table, offs)
    return out.reshape(G, D)


# speedup vs baseline: 3.3068x; 1.4838x over previous
import jax
import jax.numpy as jnp
from jax import lax
from jax.experimental import pallas as pl
from jax.experimental.pallas import tpu as pltpu
from jax.experimental.pallas import tpu_sc as plsc

N = 320000
D = 128
G = 2048
NC = 2
NS = 16
NW = NC * NS
SEG_PER_W = G // NW
T = 400
LANES = 16
NVR = D // LANES


def _wmr_body(h_hbm, pk_hbm, table_hbm, offs_hbm, out_hbm,
              ht0, ht1, pk0, pk1, acc, table_v, offs_v,
              table_s, den_s,
              sem0, sem1):
    wid = lax.axis_index("s") * NC + lax.axis_index("c")
    g0 = wid * SEG_PER_W

    pltpu.sync_copy(table_hbm, table_v)
    pltpu.sync_copy(offs_hbm, offs_v)
    tv = table_v[pl.ds(0, LANES)]
    table_s[0] = tv[0]
    table_s[1] = tv[1]
    table_s[2] = tv[2]
    ov = offs_v[pl.ds(wid, LANES)]
    r0 = ov[0]
    r1 = ov[1]

    zeros = jnp.zeros((LANES,), jnp.float32)

    def zchunk(k, carry):
        acc[pl.ds(k * LANES, LANES)] = zeros
        return carry

    lax.fori_loop(0, SEG_PER_W * D // LANES, zchunk, 0)

    def zden(l, carry):
        den_s[l] = 0.0
        return carry

    lax.fori_loop(0, SEG_PER_W, zden, 0)

    t0 = r0 // T
    t1 = (r1 + T - 1) // T
    nt = t1 - t0

    bufs = ((ht0, pk0, sem0), (ht1, pk1, sem1))

    def issue(t, buf):
        htb, pkb, sem = buf
        base = t * T
        pltpu.async_copy(h_hbm.at[pl.ds(base * D, T * D)], htb, sem)
        pltpu.async_copy(pk_hbm.at[pl.ds(base, T)], pkb, sem)

    def drain(buf):
        htb, pkb, sem = buf
        pltpu.make_async_copy(h_hbm.at[pl.ds(0, T * D)], htb, sem).wait()
        pltpu.make_async_copy(pk_hbm.at[pl.ds(0, T)], pkb, sem).wait()

    @pl.when(nt > 0)
    def _():
        issue(t0, bufs[0])

    def do_row(htb, pk, i):
        p = pk & 3
        l = (pk >> 2) - g0
        a = table_s[p]
        den_s[l] = den_s[l] + a
        ho = i * D
        ao = l * D
        for j in range(NVR):
            plsc.addupdate(acc.at[pl.ds(ao + j * LANES, LANES)],
                           a * htb[pl.ds(ho + j * LANES, LANES)])

    def process(tt, buf):
        htb, pkb, _ = buf
        base = (t0 + tt) * T
        i_lo = jnp.maximum(r0 - base, 0)
        i_hi = jnp.minimum(r1 - base, T)
        a_lo = (i_lo + LANES - 1) & ~(LANES - 1)
        a_hi = i_hi & ~(LANES - 1)
        mid_end = jnp.minimum(a_lo, i_hi)

        def row_body(i, c):
            pk = pkb[pl.ds(i, LANES)][0]
            do_row(htb, pk, i)
            return c

        def group_body(gi, c):
            ib = gi * LANES
            pkv = pkb[pl.ds(ib, LANES)]
            for lane in range(LANES):
                do_row(htb, pkv[lane], ib + lane)
            return c

        lax.fori_loop(i_lo, mid_end, row_body, 0)
        lax.fori_loop(a_lo // LANES, a_hi // LANES, group_body, 0)
        lax.fori_loop(jnp.maximum(a_hi, mid_end), i_hi, row_body, 0)

    def tile_body(tt, carry):
        for k in (0, 1):
            @pl.when((tt & 1) == k)
            def _():
                drain(bufs[k])

                @pl.when(tt + 1 < nt)
                def _():
                    issue(t0 + tt + 1, bufs[1 - k])

                process(tt, bufs[k])
        return carry

    lax.fori_loop(0, nt, tile_body, 0)

    def fin_row(l, carry):
        d = den_s[l]
        r16 = 1.0 / jnp.maximum(jnp.full((LANES,), d), 1e-12)
        for j in range(NVR):
            o = l * D + j * LANES
            acc[pl.ds(o, LANES)] = acc[pl.ds(o, LANES)] * r16
        return carry

    lax.fori_loop(0, SEG_PER_W, fin_row, 0)
    pltpu.sync_copy(acc, out_hbm.at[pl.ds(g0 * D, SEG_PER_W * D)])


_wmr = pl.kernel(
    _wmr_body,
    mesh=plsc.VectorSubcoreMesh(core_axis_name="c", subcore_axis_name="s"),
    out_type=jax.ShapeDtypeStruct((G * D,), jnp.float32),
    compiler_params=pltpu.CompilerParams(needs_layout_passes=False),
    scratch_types=[
        pltpu.VMEM((T * D,), jnp.float32),
        pltpu.VMEM((T * D,), jnp.float32),
        pltpu.VMEM((T,), jnp.int32),
        pltpu.VMEM((T,), jnp.int32),
        pltpu.VMEM((SEG_PER_W * D,), jnp.float32),
        pltpu.VMEM((LANES,), jnp.float32),
        pltpu.VMEM((NW + LANES,), jnp.int32),
        pltpu.SMEM((8,), jnp.float32),
        pltpu.SMEM((SEG_PER_W,), jnp.float32),
        pltpu.SemaphoreType.DMA,
        pltpu.SemaphoreType.DMA,
    ],
)


def kernel(h, pos, segment_ids, pos_weight):
    table = jax.nn.softplus(pos_weight[:, 0].astype(jnp.float32))
    table = jnp.pad(table, (0, LANES - table.shape[0]))
    packed = (segment_ids << 2) | pos
    bounds = jnp.arange(NW + 1, dtype=jnp.int32) * SEG_PER_W
    offs = jnp.searchsorted(segment_ids, bounds, side="left").astype(jnp.int32)
    offs = jnp.pad(offs, (0, NW + LANES - offs.shape[0]))
    out = _wmr(h.reshape(-1), packed, table, offs)
    return out.reshape(G, D)


# speedup vs baseline: 7.8026x; 2.3596x over previous
import jax
import jax.numpy as jnp
from jax import lax
from jax.experimental import pallas as pl
from jax.experimental.pallas import tpu as pltpu
from jax.experimental.pallas import tpu_sc as plsc

N = 320000
D = 128
G = 2048
NC = 2
NS = 16
NW = NC * NS
SEG_PER_W = G // NW
T = 400
LANES = 16
NVR = D // LANES


def _wmr_body(h_hbm, pk_hbm, table_hbm, offs_hbm, out_hbm,
              ht0, ht1, pk0, pk1, acc, dacc, table_v, offs_v,
              table_s,
              sem0, sem1):
    wid = lax.axis_index("s") * NC + lax.axis_index("c")
    g0 = wid * SEG_PER_W

    pltpu.sync_copy(table_hbm, table_v)
    pltpu.sync_copy(offs_hbm, offs_v)
    tv = table_v[pl.ds(0, LANES)]
    table_s[0] = tv[0]
    table_s[1] = tv[1]
    table_s[2] = tv[2]
    ov = offs_v[pl.ds(wid, LANES)]
    r0 = ov[0]
    r1 = ov[1]

    zeros = jnp.zeros((LANES,), jnp.float32)

    @plsc.parallel_loop(0, SEG_PER_W * D // LANES, unroll=8)
    def _(k):
        acc[pl.ds(k * LANES, LANES)] = zeros

    @plsc.parallel_loop(0, SEG_PER_W, unroll=8)
    def _(l):
        dacc[pl.ds(l * LANES, LANES)] = zeros

    t0 = r0 // T
    t1 = (r1 + T - 1) // T
    nt = t1 - t0

    bufs = ((ht0, pk0, sem0), (ht1, pk1, sem1))

    def issue(t, buf):
        htb, pkb, sem = buf
        base = t * T
        pltpu.async_copy(h_hbm.at[pl.ds(base * D, T * D)], htb, sem)
        pltpu.async_copy(pk_hbm.at[pl.ds(base, T)], pkb.at[pl.ds(0, T)], sem)

    def drain(buf):
        htb, pkb, sem = buf
        pltpu.make_async_copy(h_hbm.at[pl.ds(0, T * D)], htb, sem).wait()
        pltpu.make_async_copy(pk_hbm.at[pl.ds(0, T)], pkb.at[pl.ds(0, T)],
                              sem).wait()

    @pl.when(nt > 0)
    def _():
        issue(t0, bufs[0])

    def process(tt, buf):
        htb, pkb, _ = buf
        base = (t0 + tt) * T
        i_lo = jnp.maximum(r0 - base, 0)
        i_hi = jnp.minimum(r1 - base, T)

        @plsc.parallel_loop(i_lo, i_hi, unroll=8)
        def _(i):
            pk = pkb[pl.ds(i, LANES)][0]
            p = pk & 3
            l = (pk >> 2) - g0
            a = table_s[p]
            plsc.addupdate(dacc.at[pl.ds(l * LANES, LANES)],
                           jnp.full((LANES,), a))
            ho = i * D
            ao = l * D
            for j in range(NVR):
                plsc.addupdate(acc.at[pl.ds(ao + j * LANES, LANES)],
                               a * htb[pl.ds(ho + j * LANES, LANES)])

    def tile_body(tt, carry):
        for k in (0, 1):
            @pl.when((tt & 1) == k)
            def _():
                drain(bufs[k])

                @pl.when(tt + 1 < nt)
                def _():
                    issue(t0 + tt + 1, bufs[1 - k])

                process(tt, bufs[k])
        return carry

    lax.fori_loop(0, nt, tile_body, 0)

    @plsc.parallel_loop(0, SEG_PER_W, unroll=2)
    def _(l):
        r16 = 1.0 / jnp.maximum(dacc[pl.ds(l * LANES, LANES)], 1e-12)
        for j in range(NVR):
            o = l * D + j * LANES
            acc[pl.ds(o, LANES)] = acc[pl.ds(o, LANES)] * r16

    pltpu.sync_copy(acc, out_hbm.at[pl.ds(g0 * D, SEG_PER_W * D)])


_wmr = pl.kernel(
    _wmr_body,
    mesh=plsc.VectorSubcoreMesh(core_axis_name="c", subcore_axis_name="s"),
    out_type=jax.ShapeDtypeStruct((G * D,), jnp.float32),
    compiler_params=pltpu.CompilerParams(needs_layout_passes=False),
    scratch_types=[
        pltpu.VMEM((T * D,), jnp.float32),
        pltpu.VMEM((T * D,), jnp.float32),
        pltpu.VMEM((T + LANES,), jnp.int32),
        pltpu.VMEM((T + LANES,), jnp.int32),
        pltpu.VMEM((SEG_PER_W * D,), jnp.float32),
        pltpu.VMEM((SEG_PER_W * LANES,), jnp.float32),
        pltpu.VMEM((LANES,), jnp.float32),
        pltpu.VMEM((NW + LANES,), jnp.int32),
        pltpu.SMEM((8,), jnp.float32),
        pltpu.SemaphoreType.DMA,
        pltpu.SemaphoreType.DMA,
    ],
)


def kernel(h, pos, segment_ids, pos_weight):
    table = jax.nn.softplus(pos_weight[:, 0].astype(jnp.float32))
    table = jnp.pad(table, (0, LANES - table.shape[0]))
    packed = (segment_ids << 2) | pos
    bounds = jnp.arange(NW + 1, dtype=jnp.int32) * SEG_PER_W
    offs = jnp.searchsorted(segment_ids, bounds, side="left").astype(jnp.int32)
    offs = jnp.pad(offs, (0, NW + LANES - offs.shape[0]))
    out = _wmr(h.reshape(-1), packed, table, offs)
    return out.reshape(G, D)


# speedup vs baseline: 8.3995x; 1.0765x over previous
import jax
import jax.numpy as jnp
from jax import lax
from jax.experimental import pallas as pl
from jax.experimental.pallas import tpu as pltpu
from jax.experimental.pallas import tpu_sc as plsc

N = 320000
D = 128
G = 2048
NC = 2
NS = 16
NW = NC * NS
SEG_PER_W = G // NW
T = 400
LANES = 16
NVR = D // LANES


def _wmr_body(h_hbm, pk_hbm, table_hbm, offs_hbm, out_hbm,
              ht0, ht1, pk0, pk1, acc, dacc, table_v, offs_v,
              table_s,
              sem0, sem1):
    wid = lax.axis_index("s") * NC + lax.axis_index("c")
    g0d = wid * SEG_PER_W * D

    pltpu.sync_copy(table_hbm, table_v)
    pltpu.sync_copy(offs_hbm, offs_v)
    tv = table_v[pl.ds(0, LANES)]
    table_s[0] = tv[0]
    table_s[1] = tv[1]
    table_s[2] = tv[2]
    ov = offs_v[pl.ds(wid, LANES)]
    r0 = ov[0]
    r1 = ov[1]

    zeros = jnp.zeros((LANES,), jnp.float32)
    lane0_f = jnp.where(lax.iota(jnp.int32, LANES) == 0, 1.0, 0.0)

    @plsc.parallel_loop(0, SEG_PER_W * D // LANES, unroll=8)
    def _(k):
        acc[pl.ds(k * LANES, LANES)] = zeros

    @plsc.parallel_loop(0, SEG_PER_W, unroll=8)
    def _(l):
        dacc[pl.ds(l * LANES, LANES)] = zeros

    t0 = r0 // T
    t1 = (r1 + T - 1) // T
    nt = t1 - t0

    bufs = ((ht0, pk0, sem0), (ht1, pk1, sem1))

    def issue(t, buf):
        htb, pkb, sem = buf
        base = t * T
        pltpu.async_copy(h_hbm.at[pl.ds(base * D, T * D)], htb, sem)
        pltpu.async_copy(pk_hbm.at[pl.ds(base, T)], pkb.at[pl.ds(0, T)], sem)

    def drain(buf):
        htb, pkb, sem = buf
        pltpu.make_async_copy(h_hbm.at[pl.ds(0, T * D)], htb, sem).wait()
        pltpu.make_async_copy(pk_hbm.at[pl.ds(0, T)], pkb.at[pl.ds(0, T)],
                              sem).wait()

    @pl.when(nt > 0)
    def _():
        issue(t0, bufs[0])

    def do_row(htb, pk, i):
        p = pk & 3
        ao = (pk >> 2) - g0d
        a = table_s[p]
        plsc.addupdate(dacc.at[pl.ds(ao >> 3, LANES)], a * lane0_f)
        ho = i * D
        for j in range(NVR):
            plsc.addupdate(acc.at[pl.ds(ao + j * LANES, LANES)],
                           a * htb[pl.ds(ho + j * LANES, LANES)])

    def process(tt, buf):
        htb, pkb, _ = buf
        base = (t0 + tt) * T
        i_lo = jnp.maximum(r0 - base, 0)
        i_hi = jnp.minimum(r1 - base, T)
        a_lo = (i_lo + LANES - 1) & ~(LANES - 1)
        a_hi = i_hi & ~(LANES - 1)
        mid_end = jnp.minimum(a_lo, i_hi)
        tail_lo = jnp.maximum(a_hi, mid_end)
        blk_hi = jnp.maximum(a_lo, a_hi) >> 4

        @plsc.parallel_loop(i_lo, mid_end)
        def _(i):
            do_row(htb, pkb[pl.ds(i, LANES)][0], i)

        @plsc.parallel_loop(a_lo >> 4, blk_hi)
        def _(b):
            ib = b * LANES
            pkv = pkb[pl.ds(ib, LANES)]
            e0 = pkv[0]
            e15 = pkv[15]
            same = (e0 >> 9) == (e15 >> 9)

            @pl.when(same)
            def _():
                ao = (e0 >> 2) - g0d
                accs = [zeros] * NVR
                for r in range(LANES):
                    a = table_s[pkv[r] & 3]
                    ho = (ib + r) * D
                    for j in range(NVR):
                        accs[j] = accs[j] + a * htb[pl.ds(ho + j * LANES,
                                                          LANES)]
                for j in range(NVR):
                    plsc.addupdate(acc.at[pl.ds(ao + j * LANES, LANES)],
                                   accs[j])
                a16 = plsc.load_gather(table_v, [pkv & 3])
                plsc.addupdate(dacc.at[pl.ds(ao >> 3, LANES)], a16)

            @pl.when(jnp.logical_not(same))
            def _():
                for r in range(LANES):
                    do_row(htb, pkv[r], ib + r)

        @plsc.parallel_loop(tail_lo, i_hi)
        def _(i):
            do_row(htb, pkb[pl.ds(i, LANES)][0], i)

    def tile_body(tt, carry):
        for k in (0, 1):
            @pl.when((tt & 1) == k)
            def _():
                drain(bufs[k])

                @pl.when(tt + 1 < nt)
                def _():
                    issue(t0 + tt + 1, bufs[1 - k])

                process(tt, bufs[k])
        return carry

    lax.fori_loop(0, nt, tile_body, 0)

    @plsc.parallel_loop(0, SEG_PER_W, unroll=2)
    def _(l):
        d = jnp.sum(dacc[pl.ds(l * LANES, LANES)])
        r16 = 1.0 / jnp.maximum(jnp.full((LANES,), d), 1e-12)
        for j in range(NVR):
            o = l * D + j * LANES
            acc[pl.ds(o, LANES)] = acc[pl.ds(o, LANES)] * r16

    pltpu.sync_copy(acc, out_hbm.at[pl.ds(wid * SEG_PER_W * D, SEG_PER_W * D)])


_wmr = pl.kernel(
    _wmr_body,
    mesh=plsc.VectorSubcoreMesh(core_axis_name="c", subcore_axis_name="s"),
    out_type=jax.ShapeDtypeStruct((G * D,), jnp.float32),
    compiler_params=pltpu.CompilerParams(needs_layout_passes=False),
    scratch_types=[
        pltpu.VMEM((T * D,), jnp.float32),
        pltpu.VMEM((T * D,), jnp.float32),
        pltpu.VMEM((T + LANES,), jnp.int32),
        pltpu.VMEM((T + LANES,), jnp.int32),
        pltpu.VMEM((SEG_PER_W * D,), jnp.float32),
        pltpu.VMEM((SEG_PER_W * LANES,), jnp.float32),
        pltpu.VMEM((LANES,), jnp.float32),
        pltpu.VMEM((NW + LANES,), jnp.int32),
        pltpu.SMEM((8,), jnp.float32),
        pltpu.SemaphoreType.DMA,
        pltpu.SemaphoreType.DMA,
    ],
)


def kernel(h, pos, segment_ids, pos_weight):
    table = jax.nn.softplus(pos_weight[:, 0].astype(jnp.float32))
    table = jnp.pad(table, (0, LANES - table.shape[0]))
    packed = (segment_ids << 9) | pos
    bounds = jnp.arange(NW + 1, dtype=jnp.int32) * SEG_PER_W
    offs = jnp.searchsorted(segment_ids, bounds, side="left").astype(jnp.int32)
    offs = jnp.pad(offs, (0, NW + LANES - offs.shape[0]))
    out = _wmr(h.reshape(-1), packed, table, offs)
    return out.reshape(G, D)


# speedup vs baseline: 10.0816x; 1.2003x over previous
import jax
import jax.numpy as jnp
from jax import lax
from jax.experimental import pallas as pl
from jax.experimental.pallas import tpu as pltpu
from jax.experimental.pallas import tpu_sc as plsc

N = 320000
D = 128
G = 2048
NC = 2
NS = 16
NW = NC * NS
SEG_PER_W = G // NW
T = 400
LANES = 16
NVR = D // LANES


def _wmr_body(h_hbm, pk_hbm, table_hbm, offs_hbm, out_hbm,
              ht0, ht1, pk0, pk1, acc, dacc, table_v, offs_v,
              table_s,
              sem0, sem1):
    wid = lax.axis_index("s") * NC + lax.axis_index("c")
    g0d = wid * SEG_PER_W * D

    pltpu.sync_copy(table_hbm, table_v)
    pltpu.sync_copy(offs_hbm, offs_v)
    tv = table_v[pl.ds(0, LANES)]
    table_s[0] = tv[0]
    table_s[1] = tv[1]
    table_s[2] = tv[2]
    ov = offs_v[pl.ds(wid, LANES)]
    r0 = ov[0]
    r1 = ov[1]

    zeros = jnp.zeros((LANES,), jnp.float32)
    lane0_f = jnp.where(lax.iota(jnp.int32, LANES) == 0, 1.0, 0.0)

    @plsc.parallel_loop(0, SEG_PER_W * D // LANES, unroll=8)
    def _(k):
        acc[pl.ds(k * LANES, LANES)] = zeros

    @plsc.parallel_loop(0, SEG_PER_W, unroll=8)
    def _(l):
        dacc[pl.ds(l * LANES, LANES)] = zeros

    t0 = r0 // T
    t1 = (r1 + T - 1) // T
    nt = t1 - t0

    bufs = ((ht0, pk0, sem0), (ht1, pk1, sem1))

    def issue(t, buf):
        htb, pkb, sem = buf
        base = t * T
        pltpu.async_copy(h_hbm.at[pl.ds(base * D, T * D)], htb, sem)
        pltpu.async_copy(pk_hbm.at[pl.ds(base, T)], pkb.at[pl.ds(0, T)], sem)

    def drain(buf):
        htb, pkb, sem = buf
        pltpu.make_async_copy(h_hbm.at[pl.ds(0, T * D)], htb, sem).wait()
        pltpu.make_async_copy(pk_hbm.at[pl.ds(0, T)], pkb.at[pl.ds(0, T)],
                              sem).wait()

    @pl.when(nt > 0)
    def _():
        issue(t0, bufs[0])

    def do_row(htb, pk, i):
        p = pk & 3
        ao = (pk >> 2) - g0d
        a = table_s[p]
        plsc.addupdate(dacc.at[pl.ds(ao >> 3, LANES)], a * lane0_f)
        ho = i * D
        for j in range(NVR):
            plsc.addupdate(acc.at[pl.ds(ao + j * LANES, LANES)],
                           a * htb[pl.ds(ho + j * LANES, LANES)])

    def process(tt, buf):
        htb, pkb, _ = buf
        base = (t0 + tt) * T
        i_lo = jnp.maximum(r0 - base, 0)
        i_hi = jnp.minimum(r1 - base, T)
        a_lo = (i_lo + LANES - 1) & ~(LANES - 1)
        a_hi = i_hi & ~(LANES - 1)
        mid_end = jnp.minimum(a_lo, i_hi)
        tail_lo = jnp.maximum(a_hi, mid_end)
        blk_hi = jnp.maximum(a_lo, a_hi) >> 4

        @plsc.parallel_loop(i_lo, mid_end)
        def _(i):
            do_row(htb, pkb[pl.ds(i, LANES)][0], i)

        @plsc.parallel_loop(a_lo >> 4, blk_hi)
        def _(b):
            ib = b * LANES
            pkv = pkb[pl.ds(ib, LANES)]
            e0 = pkv[0]
            e15 = pkv[15]
            same = (e0 >> 9) == (e15 >> 9)

            @pl.when(same)
            def _():
                ao = (e0 >> 2) - g0d
                accs = [zeros] * NVR
                for r in range(LANES):
                    a = table_s[pkv[r] & 3]
                    ho = (ib + r) * D
                    for j in range(NVR):
                        accs[j] = accs[j] + a * htb[pl.ds(ho + j * LANES,
                                                          LANES)]
                for j in range(NVR):
                    plsc.addupdate(acc.at[pl.ds(ao + j * LANES, LANES)],
                                   accs[j])
                a16 = plsc.load_gather(table_v, [pkv & 3])
                plsc.addupdate(dacc.at[pl.ds(ao >> 3, LANES)], a16)

            @pl.when(jnp.logical_not(same))
            def _():
                seg0 = e0 >> 9
                seg15 = e15 >> 9
                aoA = (e0 >> 2) - g0d
                aoB = (e15 >> 2) - g0d
                accA = [zeros] * NVR
                accB = [zeros] * NVR
                bad = jnp.int32(0)
                for r in range(LANES):
                    pk = pkv[r]
                    seg_r = pk >> 9
                    a = table_s[pk & 3]
                    inA = seg_r == seg0
                    inB = seg_r == seg15
                    aA = jnp.where(inA, a, 0.0)
                    aB = jnp.where(inB, a, 0.0)
                    bad = bad | jnp.where(jnp.logical_or(inA, inB), 0, 1)
                    ho = (ib + r) * D
                    for j in range(NVR):
                        hv = htb[pl.ds(ho + j * LANES, LANES)]
                        accA[j] = accA[j] + aA * hv
                        accB[j] = accB[j] + aB * hv
                for j in range(NVR):
                    plsc.addupdate(acc.at[pl.ds(aoA + j * LANES, LANES)],
                                   accA[j])
                    plsc.addupdate(acc.at[pl.ds(aoB + j * LANES, LANES)],
                                   accB[j])
                svv = pkv >> 9
                a16 = plsc.load_gather(table_v, [pkv & 3])
                mA = svv == jnp.full((LANES,), seg0)
                mB = svv == jnp.full((LANES,), seg15)
                plsc.addupdate(dacc.at[pl.ds(aoA >> 3, LANES)],
                               jnp.where(mA, a16, 0.0))
                plsc.addupdate(dacc.at[pl.ds(aoB >> 3, LANES)],
                               jnp.where(mB, a16, 0.0))

                @pl.when(bad != 0)
                def _():
                    def mid_row(r, carry):
                        pk = pkb[pl.ds(ib + r, LANES)][0]
                        seg_r = pk >> 9

                        @pl.when(jnp.logical_and(seg_r != seg0,
                                                 seg_r != seg15))
                        def _():
                            do_row(htb, pk, ib + r)
                        return carry

                    lax.fori_loop(0, LANES, mid_row, 0)

        @plsc.parallel_loop(tail_lo, i_hi)
        def _(i):
            do_row(htb, pkb[pl.ds(i, LANES)][0], i)

    def tile_body(tt, carry):
        for k in (0, 1):
            @pl.when((tt & 1) == k)
            def _():
                drain(bufs[k])

                @pl.when(tt + 1 < nt)
                def _():
                    issue(t0 + tt + 1, bufs[1 - k])

                process(tt, bufs[k])
        return carry

    lax.fori_loop(0, nt, tile_body, 0)

    @plsc.parallel_loop(0, SEG_PER_W, unroll=2)
    def _(l):
        d = jnp.sum(dacc[pl.ds(l * LANES, LANES)])
        r16 = 1.0 / jnp.maximum(jnp.full((LANES,), d), 1e-12)
        for j in range(NVR):
            o = l * D + j * LANES
            acc[pl.ds(o, LANES)] = acc[pl.ds(o, LANES)] * r16

    pltpu.sync_copy(acc, out_hbm.at[pl.ds(wid * SEG_PER_W * D, SEG_PER_W * D)])


_wmr = pl.kernel(
    _wmr_body,
    mesh=plsc.VectorSubcoreMesh(core_axis_name="c", subcore_axis_name="s"),
    out_type=jax.ShapeDtypeStruct((G * D,), jnp.float32),
    compiler_params=pltpu.CompilerParams(needs_layout_passes=False),
    scratch_types=[
        pltpu.VMEM((T * D,), jnp.float32),
        pltpu.VMEM((T * D,), jnp.float32),
        pltpu.VMEM((T + LANES,), jnp.int32),
        pltpu.VMEM((T + LANES,), jnp.int32),
        pltpu.VMEM((SEG_PER_W * D,), jnp.float32),
        pltpu.VMEM((SEG_PER_W * LANES,), jnp.float32),
        pltpu.VMEM((LANES,), jnp.float32),
        pltpu.VMEM((NW + LANES,), jnp.int32),
        pltpu.SMEM((8,), jnp.float32),
        pltpu.SemaphoreType.DMA,
        pltpu.SemaphoreType.DMA,
    ],
)


def kernel(h, pos, segment_ids, pos_weight):
    table = jax.nn.softplus(pos_weight[:, 0].astype(jnp.float32))
    table = jnp.pad(table, (0, LANES - table.shape[0]))
    packed = (segment_ids << 9) | pos
    bounds = jnp.arange(NW + 1, dtype=jnp.int32) * SEG_PER_W
    offs = jnp.searchsorted(segment_ids, bounds, side="left").astype(jnp.int32)
    offs = jnp.pad(offs, (0, NW + LANES - offs.shape[0]))
    out = _wmr(h.reshape(-1), packed, table, offs)
    return out.reshape(G, D)


# speedup vs baseline: 13.9290x; 1.3816x over previous
import jax
import jax.numpy as jnp
from jax import lax
from jax.experimental import pallas as pl
from jax.experimental.pallas import tpu as pltpu
from jax.experimental.pallas import tpu_sc as plsc

N = 320000
D = 128
G = 2048
NC = 2
NS = 16
NW = NC * NS
SEG_PER_W = G // NW
T = 400
LANES = 16
NVR = D // LANES


def _wmr_body(h_hbm, pk_hbm, table_hbm, offs_hbm, out_hbm,
              ht0, ht1, pk0, pk1, acc, dacc, table_v, offs_v,
              table_s,
              sem0, sem1):
    wid = lax.axis_index("s") * NC + lax.axis_index("c")
    g0d = wid * SEG_PER_W * D

    pltpu.sync_copy(table_hbm, table_v)
    pltpu.sync_copy(offs_hbm, offs_v)
    tv = table_v[pl.ds(0, LANES)]
    table_s[0] = tv[0]
    table_s[1] = tv[1]
    table_s[2] = tv[2]
    ov = offs_v[pl.ds(wid, LANES)]
    r0 = ov[0]
    r1 = ov[1]

    zeros = jnp.zeros((LANES,), jnp.float32)
    lane0_f = jnp.where(lax.iota(jnp.int32, LANES) == 0, 1.0, 0.0)

    @plsc.parallel_loop(0, SEG_PER_W * D // LANES, unroll=8)
    def _(k):
        acc[pl.ds(k * LANES, LANES)] = zeros

    @plsc.parallel_loop(0, SEG_PER_W, unroll=8)
    def _(l):
        dacc[pl.ds(l * LANES, LANES)] = zeros

    t0 = r0 // T
    t1 = (r1 + T - 1) // T
    nt = t1 - t0

    bufs = ((ht0, pk0, sem0), (ht1, pk1, sem1))

    def issue(t, buf):
        htb, pkb, sem = buf
        base = t * T
        pltpu.async_copy(h_hbm.at[pl.ds(base * D, T * D)], htb, sem)
        pltpu.async_copy(pk_hbm.at[pl.ds(base, T)], pkb.at[pl.ds(0, T)], sem)

    def drain(buf):
        htb, pkb, sem = buf
        pltpu.make_async_copy(h_hbm.at[pl.ds(0, T * D)], htb, sem).wait()
        pltpu.make_async_copy(pk_hbm.at[pl.ds(0, T)], pkb.at[pl.ds(0, T)],
                              sem).wait()

    @pl.when(nt > 0)
    def _():
        issue(t0, bufs[0])

    def do_row(htb, pk, i):
        p = pk & 3
        ao = (pk >> 2) - g0d
        a = table_s[p]
        plsc.addupdate(dacc.at[pl.ds(ao >> 3, LANES)], a * lane0_f)
        ho = i * D
        for j in range(NVR):
            plsc.addupdate(acc.at[pl.ds(ao + j * LANES, LANES)],
                           a * htb[pl.ds(ho + j * LANES, LANES)])

    def process(tt, buf):
        htb, pkb, _ = buf
        base = (t0 + tt) * T
        i_lo = jnp.maximum(r0 - base, 0)
        i_hi = jnp.minimum(r1 - base, T)
        a_lo = (i_lo + LANES - 1) & ~(LANES - 1)
        a_hi = i_hi & ~(LANES - 1)
        mid_end = jnp.minimum(a_lo, i_hi)
        tail_lo = jnp.maximum(a_hi, mid_end)
        blk_hi = jnp.maximum(a_lo, a_hi) >> 4

        @plsc.parallel_loop(i_lo, mid_end)
        def _(i):
            do_row(htb, pkb[pl.ds(i, LANES)][0], i)

        @plsc.parallel_loop(a_lo >> 4, blk_hi)
        def _(b):
            ib = b * LANES
            pkv = pkb[pl.ds(ib, LANES)]
            e0 = pkv[0]
            e15 = pkv[15]
            same = (e0 >> 9) == (e15 >> 9)

            @pl.when(same)
            def _():
                ao = (e0 >> 2) - g0d
                accs = [zeros] * NVR
                for r in range(LANES):
                    a = table_s[pkv[r] & 3]
                    ho = (ib + r) * D
                    for j in range(NVR):
                        accs[j] = accs[j] + a * htb[pl.ds(ho + j * LANES,
                                                          LANES)]
                for j in range(NVR):
                    plsc.addupdate(acc.at[pl.ds(ao + j * LANES, LANES)],
                                   accs[j])
                a16 = plsc.load_gather(table_v, [pkv & 3])
                plsc.addupdate(dacc.at[pl.ds(ao >> 3, LANES)], a16)

            @pl.when(jnp.logical_not(same))
            def _():
                seg0 = e0 >> 9
                seg15 = e15 >> 9
                aoA = (e0 >> 2) - g0d
                aoB = (e15 >> 2) - g0d
                accA = [zeros] * NVR
                accB = [zeros] * NVR
                bad = jnp.int32(0)
                for r in range(LANES):
                    pk = pkv[r]
                    seg_r = pk >> 9
                    a = table_s[pk & 3]
                    inA = seg_r == seg0
                    inB = seg_r == seg15
                    aA = jnp.where(inA, a, 0.0)
                    aB = jnp.where(inB, a, 0.0)
                    bad = bad | jnp.where(jnp.logical_or(inA, inB), 0, 1)
                    ho = (ib + r) * D
                    for j in range(NVR):
                        hv = htb[pl.ds(ho + j * LANES, LANES)]
                        accA[j] = accA[j] + aA * hv
                        accB[j] = accB[j] + aB * hv
                for j in range(NVR):
                    plsc.addupdate(acc.at[pl.ds(aoA + j * LANES, LANES)],
                                   accA[j])
                    plsc.addupdate(acc.at[pl.ds(aoB + j * LANES, LANES)],
                                   accB[j])
                svv = pkv >> 9
                a16 = plsc.load_gather(table_v, [pkv & 3])
                mA = svv == jnp.full((LANES,), seg0)
                mB = svv == jnp.full((LANES,), seg15)
                plsc.addupdate(dacc.at[pl.ds(aoA >> 3, LANES)],
                               jnp.where(mA, a16, 0.0))
                plsc.addupdate(dacc.at[pl.ds(aoB >> 3, LANES)],
                               jnp.where(mB, a16, 0.0))

                @pl.when(bad != 0)
                def _():
                    def mid_row(r, carry):
                        pk = pkb[pl.ds(ib + r, LANES)][0]
                        seg_r = pk >> 9

                        @pl.when(jnp.logical_and(seg_r != seg0,
                                                 seg_r != seg15))
                        def _():
                            do_row(htb, pk, ib + r)
                        return carry

                    lax.fori_loop(0, LANES, mid_row, 0)

        @plsc.parallel_loop(tail_lo, i_hi)
        def _(i):
            do_row(htb, pkb[pl.ds(i, LANES)][0], i)

    def tile_body(tt, carry):
        for k in (0, 1):
            @pl.when((tt & 1) == k)
            def _():
                drain(bufs[k])

                @pl.when(tt + 1 < nt)
                def _():
                    issue(t0 + tt + 1, bufs[1 - k])

                process(tt, bufs[k])
        return carry

    lax.fori_loop(0, nt, tile_body, 0)

    @plsc.parallel_loop(0, SEG_PER_W, unroll=2)
    def _(l):
        d = jnp.sum(dacc[pl.ds(l * LANES, LANES)])
        r16 = 1.0 / jnp.maximum(jnp.full((LANES,), d), 1e-12)
        for j in range(NVR):
            o = l * D + j * LANES
            acc[pl.ds(o, LANES)] = acc[pl.ds(o, LANES)] * r16

    pltpu.sync_copy(acc, out_hbm.at[pl.ds(wid * SEG_PER_W * D, SEG_PER_W * D)])


_wmr = pl.kernel(
    _wmr_body,
    mesh=plsc.VectorSubcoreMesh(core_axis_name="c", subcore_axis_name="s"),
    out_type=jax.ShapeDtypeStruct((G * D,), jnp.float32),
    compiler_params=pltpu.CompilerParams(needs_layout_passes=False),
    scratch_types=[
        pltpu.VMEM((T * D,), jnp.float32),
        pltpu.VMEM((T * D,), jnp.float32),
        pltpu.VMEM((T + LANES,), jnp.int32),
        pltpu.VMEM((T + LANES,), jnp.int32),
        pltpu.VMEM((SEG_PER_W * D,), jnp.float32),
        pltpu.VMEM((SEG_PER_W * LANES,), jnp.float32),
        pltpu.VMEM((LANES,), jnp.float32),
        pltpu.VMEM((NW + LANES,), jnp.int32),
        pltpu.SMEM((8,), jnp.float32),
        pltpu.SemaphoreType.DMA,
        pltpu.SemaphoreType.DMA,
    ],
)


def kernel(h, pos, segment_ids, pos_weight):
    table = jax.nn.softplus(pos_weight[:, 0].astype(jnp.float32))
    table = jnp.pad(table, (0, LANES - table.shape[0]))
    packed = (segment_ids << 9) | pos
    bounds = jnp.arange(NW + 1, dtype=jnp.int32) * SEG_PER_W
    offs = jnp.sum(segment_ids[None, :] < bounds[:, None],
                   axis=1, dtype=jnp.int32)
    offs = jnp.pad(offs, (0, NW + LANES - offs.shape[0]))
    out = _wmr(h.reshape(-1), packed, table, offs)
    return out.reshape(G, D)
